# baseline (speedup 1.0000x reference)
import functools

import jax
import jax.numpy as jnp
from jax import lax
from jax.experimental import pallas as pl
from jax.experimental.pallas import tpu as pltpu
from jax.experimental.pallas import tpu_sc as plsc

_P = 7
_RPB = 16
_NW = 32


def _roi_t_kernel(roi_ref, out_ref):
    out_ref[...] = roi_ref[...].T


def _make_sc_kernel(H, W, C, NB):
    npos = _P * _P
    mesh = plsc.VectorSubcoreMesh(core_axis_name="c", subcore_axis_name="s")
    info = plsc.get_sparse_core_info()
    nc = info.num_cores
    fP = jnp.float32(_P)

    @functools.partial(
        pl.kernel,
        mesh=mesh,
        out_type=jax.ShapeDtypeStruct((NB * _RPB, npos, C), jnp.float32),
        scratch_types=[
            pltpu.VMEM((4, 16), jnp.int32),
            pltpu.VMEM((npos, 4 * 16), jnp.int32),
            pltpu.VMEM((4, npos, 16), jnp.float32),
            pltpu.VMEM((4, 4 * _RPB, C), jnp.float32),
            pltpu.VMEM((2, _RPB, C), jnp.float32),
            pltpu.SemaphoreType.DMA((4,)),
            pltpu.SemaphoreType.DMA((2,)),
        ],
        compiler_params=pltpu.CompilerParams(use_tc_tiling_on_sc=False),
    )
    def sc_kernel(feat_hbm, roit_hbm, out_hbm, roi_v, idx_v, w_v, gbuf, obuf,
                  gsem, osem):
        wid = lax.axis_index("s") * nc + lax.axis_index("c")
        nblk = (NB - wid + _NW - 1) // _NW

        def fire(p, slot):
            pltpu.async_copy(feat_hbm.at[idx_v.at[p]], gbuf.at[slot],
                             gsem.at[slot])

        def drain(p, slot):
            pltpu.make_async_copy(feat_hbm.at[idx_v.at[p]], gbuf.at[slot],
                                  gsem.at[slot]).wait()

        def outer(i, carry):
            b = wid + i * _NW
            base = b * _RPB
            pltpu.sync_copy(roit_hbm.at[:, pl.ds(base, _RPB)], roi_v)
            y1 = roi_v[0]
            x1 = roi_v[1]
            y2 = roi_v[2]
            x2 = roi_v[3]
            one = jnp.float32(1.0)

            wd = jnp.maximum(y2 - y1, 1)
            wf7 = wd.astype(jnp.float32) / fP
            c0l, c1l, cfl, cf1l = [], [], [], []
            for j in range(_P):
                cpos = jnp.float32(j) * wf7
                c0 = cpos.astype(jnp.int32)
                c1 = jnp.minimum(c0 + 1, wd - 1)
                cf = cpos - c0.astype(jnp.float32)
                c0l.append(jnp.clip(y1 + c0, 0, W - 1))
                c1l.append(jnp.clip(y1 + c1, 0, W - 1))
                cfl.append(cf)
                cf1l.append(one - cf)

            h = jnp.maximum(x2 - x1, 1)
            hf7 = h.astype(jnp.float32) / fP
            for i in range(_P):
                rpos = jnp.float32(i) * hf7
                r0 = rpos.astype(jnp.int32)
                r1 = jnp.minimum(r0 + 1, h - 1)
                rf = rpos - r0.astype(jnp.float32)
                rf1 = one - rf
                R0W = jnp.clip(x1 + r0, 0, H - 1) * W
                R1W = jnp.clip(x1 + r1, 0, H - 1) * W
                for j in range(_P):
                    p = i * _P + j
                    idx_v[p, pl.ds(0, 16)] = R0W + c0l[j]
                    idx_v[p, pl.ds(16, 16)] = R0W + c1l[j]
                    idx_v[p, pl.ds(32, 16)] = R1W + c0l[j]
                    idx_v[p, pl.ds(48, 16)] = R1W + c1l[j]
                    w_v[0, p] = rf1 * cf1l[j]
                    w_v[1, p] = rf1 * cfl[j]
                    w_v[2, p] = rf * cf1l[j]
                    w_v[3, p] = rf * cfl[j]

            fire(0, 0)
            fire(1, 1)
            fire(2, 2)

            def inner(p, carry2):
                slot = lax.rem(p, 4)
                oslot = lax.rem(p, 2)

                @pl.when(p + 3 < npos)
                def _prefetch():
                    fire(p + 3, lax.rem(p + 3, 4))

                drain(p, slot)

                @pl.when(p >= 2)
                def _owait():
                    pltpu.make_async_copy(
                        obuf.at[oslot],
                        out_hbm.at[pl.ds(base, _RPB), p - 2],
                        osem.at[oslot]).wait()

                for r in range(_RPB):
                    w00 = w_v[0, p][r]
                    w01 = w_v[1, p][r]
                    w10 = w_v[2, p][r]
                    w11 = w_v[3, p][r]
                    for c in range(C // 16):
                        cs = pl.ds(c * 16, 16)
                        obuf[oslot, r, cs] = (
                            w00 * gbuf[slot, r, cs]
                            + w01 * gbuf[slot, _RPB + r, cs]
                            + w10 * gbuf[slot, 2 * _RPB + r, cs]
                            + w11 * gbuf[slot, 3 * _RPB + r, cs])

                pltpu.async_copy(obuf.at[oslot],
                                 out_hbm.at[pl.ds(base, _RPB), p],
                                 osem.at[oslot])
                return carry2

            lax.fori_loop(0, npos, inner, 0)
            for p in (npos - 2, npos - 1):
                pltpu.make_async_copy(
                    obuf.at[p % 2],
                    out_hbm.at[pl.ds(base, _RPB), p],
                    osem.at[p % 2]).wait()
            return carry

        lax.fori_loop(0, nblk, outer, 0)

    return sc_kernel


def kernel(features, roi):
    B, H, W, C = features.shape
    N = roi.shape[0] * roi.shape[1]
    roi2 = roi.reshape(N, 4).astype(jnp.int32)
    feat2 = features.reshape(B * H * W, C)

    roit = pl.pallas_call(
        _roi_t_kernel,
        out_shape=jax.ShapeDtypeStruct((4, N), jnp.int32),
    )(roi2)

    NB = N // _RPB
    out = _make_sc_kernel(H, W, C, NB)(feat2, roit)
    return out.reshape(N, _P, _P, C)


# speedup vs baseline: 1.0052x; 1.0052x over previous
import functools

import jax
import jax.numpy as jnp
from jax import lax
from jax.experimental import pallas as pl
from jax.experimental.pallas import tpu as pltpu
from jax.experimental.pallas import tpu_sc as plsc

_P = 7
_RPB = 16
_NW = 32


def _roi_t_kernel(roi_ref, out_ref):
    out_ref[...] = roi_ref[...].T


def _make_sc_kernel(H, W, C, NB):
    npos = _P * _P
    mesh = plsc.VectorSubcoreMesh(core_axis_name="c", subcore_axis_name="s")
    info = plsc.get_sparse_core_info()
    nc = info.num_cores
    fP = jnp.float32(_P)

    @functools.partial(
        pl.kernel,
        mesh=mesh,
        out_type=jax.ShapeDtypeStruct((NB * _RPB, npos, C), jnp.float32),
        scratch_types=[
            pltpu.VMEM((4, 16), jnp.int32),
            pltpu.VMEM((npos, 4 * 16), jnp.int32),
            pltpu.VMEM((4, npos, 16), jnp.float32),
            pltpu.VMEM((2, 4 * _RPB, C), jnp.float32),
            pltpu.VMEM((2, _RPB, C), jnp.float32),
            pltpu.SemaphoreType.DMA((2,)),
            pltpu.SemaphoreType.DMA((2,)),
        ],
        compiler_params=pltpu.CompilerParams(use_tc_tiling_on_sc=False),
    )
    def sc_kernel(feat_hbm, roit_hbm, out_hbm, roi_v, idx_v, w_v, gbuf, obuf,
                  gsem, osem):
        wid = lax.axis_index("s") * nc + lax.axis_index("c")
        nblk = (NB - wid + _NW - 1) // _NW

        def fire(p, slot):
            pltpu.async_copy(feat_hbm.at[idx_v.at[p]], gbuf.at[slot],
                             gsem.at[slot])

        def drain(p, slot):
            pltpu.make_async_copy(feat_hbm.at[idx_v.at[p]], gbuf.at[slot],
                                  gsem.at[slot]).wait()

        def outer(i, carry):
            b = wid + i * _NW
            base = b * _RPB
            pltpu.sync_copy(roit_hbm.at[:, pl.ds(base, _RPB)], roi_v)
            y1 = roi_v[0]
            x1 = roi_v[1]
            y2 = roi_v[2]
            x2 = roi_v[3]
            one = jnp.float32(1.0)

            wd = jnp.maximum(y2 - y1, 1)
            wf7 = wd.astype(jnp.float32) / fP
            c0l, c1l, cfl, cf1l = [], [], [], []
            for j in range(_P):
                cpos = jnp.float32(j) * wf7
                c0 = cpos.astype(jnp.int32)
                c1 = jnp.minimum(c0 + 1, wd - 1)
                cf = cpos - c0.astype(jnp.float32)
                c0l.append(jnp.clip(y1 + c0, 0, W - 1))
                c1l.append(jnp.clip(y1 + c1, 0, W - 1))
                cfl.append(cf)
                cf1l.append(one - cf)

            h = jnp.maximum(x2 - x1, 1)
            hf7 = h.astype(jnp.float32) / fP
            for i in range(_P):
                rpos = jnp.float32(i) * hf7
                r0 = rpos.astype(jnp.int32)
                r1 = jnp.minimum(r0 + 1, h - 1)
                rf = rpos - r0.astype(jnp.float32)
                rf1 = one - rf
                R0W = jnp.clip(x1 + r0, 0, H - 1) * W
                R1W = jnp.clip(x1 + r1, 0, H - 1) * W
                for j in range(_P):
                    p = i * _P + j
                    idx_v[p, pl.ds(0, 16)] = R0W + c0l[j]
                    idx_v[p, pl.ds(16, 16)] = R0W + c1l[j]
                    idx_v[p, pl.ds(32, 16)] = R1W + c0l[j]
                    idx_v[p, pl.ds(48, 16)] = R1W + c1l[j]
                    w_v[0, p] = rf1 * cf1l[j]
                    w_v[1, p] = rf1 * cfl[j]
                    w_v[2, p] = rf * cf1l[j]
                    w_v[3, p] = rf * cfl[j]

            fire(0, 0)

            def inner(p, carry2):
                slot = lax.rem(p, 2)
                nslot = lax.rem(p + 1, 2)

                @pl.when(p + 1 < npos)
                def _prefetch():
                    fire(p + 1, nslot)

                drain(p, slot)

                @pl.when(p >= 2)
                def _owait():
                    pltpu.make_async_copy(
                        obuf.at[slot],
                        out_hbm.at[pl.ds(base, _RPB), p - 2],
                        osem.at[slot]).wait()

                for r in range(_RPB):
                    w00 = w_v[0, p][r]
                    w01 = w_v[1, p][r]
                    w10 = w_v[2, p][r]
                    w11 = w_v[3, p][r]
                    for c in range(C // 16):
                        cs = pl.ds(c * 16, 16)
                        obuf[slot, r, cs] = (
                            w00 * gbuf[slot, r, cs]
                            + w01 * gbuf[slot, _RPB + r, cs]
                            + w10 * gbuf[slot, 2 * _RPB + r, cs]
                            + w11 * gbuf[slot, 3 * _RPB + r, cs])

                pltpu.async_copy(obuf.at[slot],
                                 out_hbm.at[pl.ds(base, _RPB), p],
                                 osem.at[slot])
                return carry2

            lax.fori_loop(0, npos, inner, 0)
            for p in (npos - 2, npos - 1):
                pltpu.make_async_copy(
                    obuf.at[p % 2],
                    out_hbm.at[pl.ds(base, _RPB), p],
                    osem.at[p % 2]).wait()
            return carry

        lax.fori_loop(0, nblk, outer, 0)

    return sc_kernel


def kernel(features, roi):
    B, H, W, C = features.shape
    N = roi.shape[0] * roi.shape[1]
    roi2 = roi.reshape(N, 4).astype(jnp.int32)
    feat2 = features.reshape(B * H * W, C)

    roit = pl.pallas_call(
        _roi_t_kernel,
        out_shape=jax.ShapeDtypeStruct((4, N), jnp.int32),
    )(roi2)

    NB = N // _RPB
    out = _make_sc_kernel(H, W, C, NB)(feat2, roit)
    return out.reshape(N, _P, _P, C)


# speedup vs baseline: 1.0169x; 1.0117x over previous
import functools

import jax
import jax.numpy as jnp
from jax import lax
from jax.experimental import pallas as pl
from jax.experimental.pallas import tpu as pltpu
from jax.experimental.pallas import tpu_sc as plsc

_P = 7
_RPB = 16
_NW = 32


def _roi_t_kernel(roi_ref, out_ref):
    out_ref[...] = roi_ref[...].T


def _make_sc_kernel(H, W, C, NB):
    npos = _P * _P
    mesh = plsc.VectorSubcoreMesh(core_axis_name="c", subcore_axis_name="s")
    info = plsc.get_sparse_core_info()
    nc = info.num_cores
    fP = jnp.float32(_P)

    @functools.partial(
        pl.kernel,
        mesh=mesh,
        out_type=jax.ShapeDtypeStruct((NB * _RPB, npos, C), jnp.float32),
        scratch_types=[
            pltpu.VMEM((4, 16), jnp.int32),
            pltpu.VMEM((npos, 4 * 16), jnp.int32),
            pltpu.VMEM((4, npos, 16), jnp.float32),
            pltpu.VMEM((2, 4 * _RPB, C), jnp.float32),
            pltpu.VMEM((2, _RPB, C), jnp.float32),
            pltpu.SemaphoreType.DMA((2,)),
            pltpu.SemaphoreType.DMA((2,)),
        ],
        compiler_params=pltpu.CompilerParams(use_tc_tiling_on_sc=False),
    )
    def sc_kernel(feat_hbm, roit_hbm, out_hbm, roi_v, idx_v, w_v, gbuf, obuf,
                  gsem, osem):
        wid = lax.axis_index("s") * nc + lax.axis_index("c")
        nblk = (NB - wid + _NW - 1) // _NW

        def fire(p, slot):
            pltpu.async_copy(feat_hbm.at[idx_v.at[p]], gbuf.at[slot],
                             gsem.at[slot])

        def drain(p, slot):
            pltpu.make_async_copy(feat_hbm.at[idx_v.at[p]], gbuf.at[slot],
                                  gsem.at[slot]).wait()

        def outer(i, carry):
            b = wid + i * _NW
            base = b * _RPB
            pltpu.sync_copy(roit_hbm.at[:, pl.ds(base, _RPB)], roi_v)
            y1 = roi_v[0]
            x1 = roi_v[1]
            y2 = roi_v[2]
            x2 = roi_v[3]
            one = jnp.float32(1.0)

            wd = jnp.maximum(y2 - y1, 1)
            wf7 = wd.astype(jnp.float32) / fP
            c0l, c1l, cfl, cf1l = [], [], [], []
            for j in range(_P):
                cpos = jnp.float32(j) * wf7
                c0 = cpos.astype(jnp.int32)
                c1 = jnp.minimum(c0 + 1, wd - 1)
                cf = cpos - c0.astype(jnp.float32)
                c0l.append(jnp.clip(y1 + c0, 0, W - 1))
                c1l.append(jnp.clip(y1 + c1, 0, W - 1))
                cfl.append(cf)
                cf1l.append(one - cf)

            h = jnp.maximum(x2 - x1, 1)
            hf7 = h.astype(jnp.float32) / fP
            for i in range(_P):
                rpos = jnp.float32(i) * hf7
                r0 = rpos.astype(jnp.int32)
                r1 = jnp.minimum(r0 + 1, h - 1)
                rf = rpos - r0.astype(jnp.float32)
                rf1 = one - rf
                R0W = jnp.clip(x1 + r0, 0, H - 1) * W
                R1W = jnp.clip(x1 + r1, 0, H - 1) * W
                for j in range(_P):
                    p = i * _P + j
                    idx_v[p, pl.ds(0, 16)] = R0W + c0l[j]
                    idx_v[p, pl.ds(16, 16)] = R0W + c1l[j]
                    idx_v[p, pl.ds(32, 16)] = R1W + c0l[j]
                    idx_v[p, pl.ds(48, 16)] = R1W + c1l[j]
                    w_v[0, p] = rf1 * cf1l[j]
                    w_v[1, p] = rf1 * cfl[j]
                    w_v[2, p] = rf * cf1l[j]
                    w_v[3, p] = rf * cfl[j]

            fire(0, 0)

            def inner(p, carry2):
                slot = lax.rem(p, 2)
                nslot = lax.rem(p + 1, 2)

                @pl.when(p + 1 < npos)
                def _prefetch():
                    fire(p + 1, nslot)

                drain(p, slot)

                @pl.when(p >= 2)
                def _owait():
                    pltpu.make_async_copy(
                        obuf.at[slot],
                        out_hbm.at[pl.ds(base, _RPB), p - 2],
                        osem.at[slot]).wait()

                wv00 = w_v[0, p]
                wv01 = w_v[1, p]
                wv10 = w_v[2, p]
                wv11 = w_v[3, p]
                for r in range(_RPB):
                    w00 = wv00[r]
                    w01 = wv01[r]
                    w10 = wv10[r]
                    w11 = wv11[r]
                    for c in range(C // 16):
                        cs = pl.ds(c * 16, 16)
                        obuf[slot, r, cs] = (
                            w00 * gbuf[slot, r, cs]
                            + w01 * gbuf[slot, _RPB + r, cs]
                            + w10 * gbuf[slot, 2 * _RPB + r, cs]
                            + w11 * gbuf[slot, 3 * _RPB + r, cs])

                pltpu.async_copy(obuf.at[slot],
                                 out_hbm.at[pl.ds(base, _RPB), p],
                                 osem.at[slot])
                return carry2

            lax.fori_loop(0, npos, inner, 0)
            for p in (npos - 2, npos - 1):
                pltpu.make_async_copy(
                    obuf.at[p % 2],
                    out_hbm.at[pl.ds(base, _RPB), p],
                    osem.at[p % 2]).wait()
            return carry

        lax.fori_loop(0, nblk, outer, 0)

    return sc_kernel


def kernel(features, roi):
    B, H, W, C = features.shape
    N = roi.shape[0] * roi.shape[1]
    roi2 = roi.reshape(N, 4).astype(jnp.int32)
    feat2 = features.reshape(B * H * W, C)

    roit = pl.pallas_call(
        _roi_t_kernel,
        out_shape=jax.ShapeDtypeStruct((4, N), jnp.int32),
    )(roi2)

    NB = N // _RPB
    out = _make_sc_kernel(H, W, C, NB)(feat2, roit)
    return out.reshape(N, _P, _P, C)


# speedup vs baseline: 1.1815x; 1.1618x over previous
import functools

import jax
import jax.numpy as jnp
from jax import lax
from jax.experimental import pallas as pl
from jax.experimental.pallas import tpu as pltpu
from jax.experimental.pallas import tpu_sc as plsc

_P = 7
_RPB = 16
_NW = 32


def _roi_t_kernel(roi_ref, out_ref):
    out_ref[...] = roi_ref[...].T


def _make_sc_kernel(H, W, C, NB):
    npos = _P * _P
    mesh = plsc.VectorSubcoreMesh(core_axis_name="c", subcore_axis_name="s")
    info = plsc.get_sparse_core_info()
    nc = info.num_cores
    fP = jnp.float32(_P)

    @functools.partial(
        pl.kernel,
        mesh=mesh,
        out_type=jax.ShapeDtypeStruct((NB * _RPB, npos, C), jnp.float32),
        scratch_types=[
            pltpu.VMEM((4, 16), jnp.int32),
            pltpu.VMEM((npos, 4 * 16), jnp.int32),
            pltpu.VMEM((4, npos, 16), jnp.float32),
            pltpu.VMEM((2, 4 * _RPB, C), jnp.float32),
            pltpu.VMEM((2, _RPB, C), jnp.float32),
            pltpu.SemaphoreType.DMA((2,)),
            pltpu.SemaphoreType.DMA((2,)),
        ],
        compiler_params=pltpu.CompilerParams(use_tc_tiling_on_sc=False),
    )
    def sc_kernel(feat_hbm, roit_hbm, out_hbm, roi_v, idx_v, w_v, gbuf, obuf,
                  gsem, osem):
        wid = lax.axis_index("s") * nc + lax.axis_index("c")
        nblk = (NB - wid + _NW - 1) // _NW

        def fire(p, slot):
            pltpu.async_copy(feat_hbm.at[idx_v.at[p]], gbuf.at[slot],
                             gsem.at[slot])

        def drain(p, slot):
            pltpu.make_async_copy(feat_hbm.at[idx_v.at[p]], gbuf.at[slot],
                                  gsem.at[slot]).wait()

        def outer(i, carry):
            b = wid + i * _NW
            base = b * _RPB
            pltpu.sync_copy(roit_hbm.at[:, pl.ds(base, _RPB)], roi_v)
            y1 = roi_v[0]
            x1 = roi_v[1]
            y2 = roi_v[2]
            x2 = roi_v[3]
            one = jnp.float32(1.0)

            wd = jnp.maximum(y2 - y1, 1)
            wf7 = wd.astype(jnp.float32) / fP
            c0l, c1l, cfl, cf1l = [], [], [], []
            for j in range(_P):
                cpos = jnp.float32(j) * wf7
                c0 = cpos.astype(jnp.int32)
                c1 = jnp.minimum(c0 + 1, wd - 1)
                cf = cpos - c0.astype(jnp.float32)
                c0l.append(jnp.clip(y1 + c0, 0, W - 1))
                c1l.append(jnp.clip(y1 + c1, 0, W - 1))
                cfl.append(cf)
                cf1l.append(one - cf)

            h = jnp.maximum(x2 - x1, 1)
            hf7 = h.astype(jnp.float32) / fP
            for i in range(_P):
                rpos = jnp.float32(i) * hf7
                r0 = rpos.astype(jnp.int32)
                r1 = jnp.minimum(r0 + 1, h - 1)
                rf = rpos - r0.astype(jnp.float32)
                rf1 = one - rf
                R0W = jnp.clip(x1 + r0, 0, H - 1) * W
                R1W = jnp.clip(x1 + r1, 0, H - 1) * W
                for j in range(_P):
                    p = i * _P + j
                    idx_v[p, pl.ds(0, 16)] = R0W + c0l[j]
                    idx_v[p, pl.ds(16, 16)] = R0W + c1l[j]
                    idx_v[p, pl.ds(32, 16)] = R1W + c0l[j]
                    idx_v[p, pl.ds(48, 16)] = R1W + c1l[j]
                    w_v[0, p] = rf1 * cf1l[j]
                    w_v[1, p] = rf1 * cfl[j]
                    w_v[2, p] = rf * cf1l[j]
                    w_v[3, p] = rf * cfl[j]

            fire(0, 0)

            def step(p, slot, do_owait):
                drain(p, slot)

                @pl.when(do_owait)
                def _owait():
                    pltpu.make_async_copy(
                        obuf.at[slot],
                        out_hbm.at[pl.ds(base, _RPB), p - 2],
                        osem.at[slot]).wait()

                wv00 = w_v[0, p]
                wv01 = w_v[1, p]
                wv10 = w_v[2, p]
                wv11 = w_v[3, p]
                for r in range(_RPB):
                    w00 = wv00[r]
                    w01 = wv01[r]
                    w10 = wv10[r]
                    w11 = wv11[r]
                    for c in range(C // 16):
                        cs = pl.ds(c * 16, 16)
                        obuf[slot, r, cs] = (
                            w00 * gbuf[slot, r, cs]
                            + w01 * gbuf[slot, _RPB + r, cs]
                            + w10 * gbuf[slot, 2 * _RPB + r, cs]
                            + w11 * gbuf[slot, 3 * _RPB + r, cs])

                pltpu.async_copy(obuf.at[slot],
                                 out_hbm.at[pl.ds(base, _RPB), p],
                                 osem.at[slot])

            def inner(t, carry2):
                p0 = 2 * t
                fire(p0 + 1, 1)
                step(p0, 0, t >= 1)
                fire(p0 + 2, 0)
                step(p0 + 1, 1, t >= 1)
                return carry2

            lax.fori_loop(0, (npos - 1) // 2, inner, 0)
            step(npos - 1, 0, jnp.bool_(True))
            for p in (npos - 2, npos - 1):
                pltpu.make_async_copy(
                    obuf.at[p % 2],
                    out_hbm.at[pl.ds(base, _RPB), p],
                    osem.at[p % 2]).wait()
            return carry

        lax.fori_loop(0, nblk, outer, 0)

    return sc_kernel


def kernel(features, roi):
    B, H, W, C = features.shape
    N = roi.shape[0] * roi.shape[1]
    roi2 = roi.reshape(N, 4).astype(jnp.int32)
    feat2 = features.reshape(B * H * W, C)

    roit = pl.pallas_call(
        _roi_t_kernel,
        out_shape=jax.ShapeDtypeStruct((4, N), jnp.int32),
    )(roi2)

    NB = N // _RPB
    out = _make_sc_kernel(H, W, C, NB)(feat2, roit)
    return out.reshape(N, _P, _P, C)
